# Initial kernel scaffold; baseline (speedup 1.0000x reference)
import jax
import jax.numpy as jnp
from jax.experimental import pallas as pl


def kernel(x, edge_index, W0, b0, W1, b1):
    raise NotImplementedError("write your pallas kernel here")


# speedup vs baseline: 5.3280x; 5.3280x over previous
import functools

import jax
import jax.numpy as jnp
from jax import lax
from jax.experimental import pallas as pl
from jax.experimental.pallas import tpu as pltpu
from jax.experimental.pallas import tpu_sc as plsc

NC = 2
NS = 16
NW = NC * NS

_CHUNK = 80


@functools.lru_cache(maxsize=None)
def _make_segsum(n, e, d):
    assert e % (NW * _CHUNK) == 0
    e_per = e // NW
    nch = e_per // _CHUNK
    rps = (n // NS) // 8 * 8
    rlast = n - (NS - 1) * rps
    mesh = plsc.VectorSubcoreMesh(
        core_axis_name="c", subcore_axis_name="s", num_cores=NC, num_subcores=NS
    )

    @functools.partial(
        pl.kernel,
        mesh=mesh,
        out_type=jax.ShapeDtypeStruct((NC, n, d), jnp.float32),
        scratch_types=[
            pltpu.VMEM((_CHUNK,), jnp.int32),
            pltpu.VMEM((_CHUNK,), jnp.int32),
            pltpu.VMEM((_CHUNK, d), jnp.float32),
            pltpu.VMEM_SHARED((n, d), jnp.float32),
            pltpu.SemaphoreType.DMA,
        ],
        compiler_params=pltpu.CompilerParams(use_tc_tiling_on_sc=False),
    )
    def segsum(x_hbm, src_hbm, dst_hbm, zeros_hbm, out_hbm,
               src_v, dst_v, rows_v, acc, sem):
        c = lax.axis_index("c")
        s = lax.axis_index("s")
        wid = s * NC + c
        row0 = s * rps

        @pl.when(s < NS - 1)
        def _():
            pltpu.sync_copy(zeros_hbm.at[pl.ds(row0, rps)],
                            acc.at[pl.ds(row0, rps)])

        @pl.when(s == NS - 1)
        def _():
            pltpu.sync_copy(zeros_hbm.at[pl.ds((NS - 1) * rps, rlast)],
                            acc.at[pl.ds((NS - 1) * rps, rlast)])

        plsc.subcore_barrier()
        base = wid * e_per

        def body(i, carry):
            off = base + i * _CHUNK
            pltpu.sync_copy(src_hbm.at[pl.ds(off, _CHUNK)], src_v)
            pltpu.sync_copy(dst_hbm.at[pl.ds(off, _CHUNK)], dst_v)
            pltpu.async_copy(x_hbm.at[src_v], rows_v, sem).wait()
            pltpu.sync_copy(rows_v, acc.at[dst_v], add=True)
            return carry

        lax.fori_loop(0, nch, body, 0)
        plsc.subcore_barrier()

        @pl.when(s < NS - 1)
        def _():
            pltpu.sync_copy(acc.at[pl.ds(row0, rps)],
                            out_hbm.at[c, pl.ds(row0, rps)])

        @pl.when(s == NS - 1)
        def _():
            pltpu.sync_copy(acc.at[pl.ds((NS - 1) * rps, rlast)],
                            out_hbm.at[c, pl.ds((NS - 1) * rps, rlast)])

    return segsum


def _mid_body(p_ref, w0_ref, b0_ref, w1_ref, o_ref):
    ssum = p_ref[0] + p_ref[1]
    h = jnp.dot(ssum, w0_ref[...], preferred_element_type=jnp.float32)
    h = jnp.maximum(h + b0_ref[...], 0.0)
    o_ref[...] = jnp.dot(h, w1_ref[...], preferred_element_type=jnp.float32)


def _final_body(p_ref, b1_ref, o_ref):
    ssum = p_ref[0] + p_ref[1] + b1_ref[...]
    m = jnp.max(ssum, axis=1, keepdims=True)
    shifted = ssum - m
    o_ref[...] = shifted - jnp.log(jnp.sum(jnp.exp(shifted), axis=1, keepdims=True))


def kernel(x, edge_index, W0, b0, W1, b1):
    n, d = x.shape
    h_dim = W0.shape[1]
    o_dim = W1.shape[1]
    e = edge_index.shape[1]
    src = edge_index[0]
    dst = edge_index[1]

    zeros_d = jnp.zeros((n, d), jnp.float32)
    p1 = _make_segsum(n, e, d)(x, src, dst, zeros_d)

    blk = 1000
    hw = pl.pallas_call(
        _mid_body,
        grid=(n // blk,),
        in_specs=[
            pl.BlockSpec((NC, blk, d), lambda i: (0, i, 0)),
            pl.BlockSpec((d, h_dim), lambda i: (0, 0)),
            pl.BlockSpec((1, h_dim), lambda i: (0, 0)),
            pl.BlockSpec((h_dim, o_dim), lambda i: (0, 0)),
        ],
        out_specs=pl.BlockSpec((blk, o_dim), lambda i: (i, 0)),
        out_shape=jax.ShapeDtypeStruct((n, o_dim), jnp.float32),
    )(p1, W0, b0.reshape(1, h_dim), W1)

    zeros_o = jnp.zeros((n, o_dim), jnp.float32)
    p2 = _make_segsum(n, e, o_dim)(hw, src, dst, zeros_o)

    out = pl.pallas_call(
        _final_body,
        grid=(n // blk,),
        in_specs=[
            pl.BlockSpec((NC, blk, o_dim), lambda i: (0, i, 0)),
            pl.BlockSpec((1, o_dim), lambda i: (0, 0)),
        ],
        out_specs=pl.BlockSpec((blk, o_dim), lambda i: (i, 0)),
        out_shape=jax.ShapeDtypeStruct((n, o_dim), jnp.float32),
    )(p2, b1.reshape(1, o_dim))
    return out


# speedup vs baseline: 12.9861x; 2.4373x over previous
import functools

import jax
import jax.numpy as jnp
from jax import lax
from jax.experimental import pallas as pl
from jax.experimental.pallas import tpu as pltpu
from jax.experimental.pallas import tpu_sc as plsc

NC = 2
NS = 16
NW = NC * NS

_CHUNK = 40


_NBUF = 5


@functools.lru_cache(maxsize=None)
def _make_segsum(n, e, d):
    assert e % (NW * _CHUNK) == 0
    e_per = e // NW
    nch = e_per // _CHUNK
    assert nch % _NBUF == 0
    nit = nch // _NBUF
    rps = (n // NS) // 8 * 8
    rlast = n - (NS - 1) * rps
    mesh = plsc.VectorSubcoreMesh(
        core_axis_name="c", subcore_axis_name="s", num_cores=NC, num_subcores=NS
    )

    @functools.partial(
        pl.kernel,
        mesh=mesh,
        out_type=jax.ShapeDtypeStruct((NC, n, d), jnp.float32),
        scratch_types=[
            pltpu.VMEM((nch, _CHUNK), jnp.int32),
            pltpu.VMEM((nch, _CHUNK), jnp.int32),
            [pltpu.VMEM((_CHUNK, d), jnp.float32) for _ in range(_NBUF)],
            pltpu.VMEM_SHARED((n, d), jnp.float32),
            [pltpu.SemaphoreType.DMA for _ in range(_NBUF)],
            [pltpu.SemaphoreType.DMA for _ in range(_NBUF)],
        ],
        compiler_params=pltpu.CompilerParams(use_tc_tiling_on_sc=False),
    )
    def segsum(x_hbm, src_hbm, dst_hbm, zeros_hbm, out_hbm,
               src_v, dst_v, rows, acc, sem_g, sem_s):
        c = lax.axis_index("c")
        s = lax.axis_index("s")
        wid = s * NC + c
        row0 = s * rps

        pltpu.sync_copy(src_hbm.at[pl.ds(wid * nch, nch)], src_v)
        pltpu.sync_copy(dst_hbm.at[pl.ds(wid * nch, nch)], dst_v)

        @pl.when(s < NS - 1)
        def _():
            pltpu.sync_copy(zeros_hbm.at[pl.ds(row0, rps)],
                            acc.at[pl.ds(row0, rps)])

        @pl.when(s == NS - 1)
        def _():
            pltpu.sync_copy(zeros_hbm.at[pl.ds((NS - 1) * rps, rlast)],
                            acc.at[pl.ds((NS - 1) * rps, rlast)])

        plsc.subcore_barrier()

        def body(k, carry):
            j0 = k * _NBUF
            for b in range(_NBUF):
                @pl.when(k > 0)
                def _():
                    pltpu.make_async_copy(
                        rows[b], acc.at[pl.ds(0, _CHUNK)], sem_s[b]).wait()
                pltpu.async_copy(x_hbm.at[src_v.at[j0 + b]], rows[b], sem_g[b])
            for b in range(_NBUF):
                pltpu.make_async_copy(
                    x_hbm.at[pl.ds(0, _CHUNK)], rows[b], sem_g[b]).wait()
                pltpu.async_copy(rows[b], acc.at[dst_v.at[j0 + b]],
                                 sem_s[b], add=True)
            return carry

        lax.fori_loop(0, nit, body, 0)
        for b in range(_NBUF):
            pltpu.make_async_copy(
                rows[b], acc.at[pl.ds(0, _CHUNK)], sem_s[b]).wait()
        plsc.subcore_barrier()

        @pl.when(s < NS - 1)
        def _():
            pltpu.sync_copy(acc.at[pl.ds(row0, rps)],
                            out_hbm.at[c, pl.ds(row0, rps)])

        @pl.when(s == NS - 1)
        def _():
            pltpu.sync_copy(acc.at[pl.ds((NS - 1) * rps, rlast)],
                            out_hbm.at[c, pl.ds((NS - 1) * rps, rlast)])

    return segsum


def _mid_body(p_ref, w0_ref, b0_ref, w1_ref, o_ref):
    ssum = p_ref[0] + p_ref[1]
    h = jnp.dot(ssum, w0_ref[...], preferred_element_type=jnp.float32)
    h = jnp.maximum(h + b0_ref[...], 0.0)
    o_ref[...] = jnp.dot(h, w1_ref[...], preferred_element_type=jnp.float32)


def _final_body(p_ref, b1_ref, o_ref):
    ssum = p_ref[0] + p_ref[1] + b1_ref[...]
    m = jnp.max(ssum, axis=1, keepdims=True)
    shifted = ssum - m
    o_ref[...] = shifted - jnp.log(jnp.sum(jnp.exp(shifted), axis=1, keepdims=True))


def kernel(x, edge_index, W0, b0, W1, b1):
    n, d = x.shape
    h_dim = W0.shape[1]
    o_dim = W1.shape[1]
    e = edge_index.shape[1]
    src = edge_index[0].reshape(-1, _CHUNK)
    dst = edge_index[1].reshape(-1, _CHUNK)

    zeros_d = jnp.zeros((n, d), jnp.float32)
    p1 = _make_segsum(n, e, d)(x, src, dst, zeros_d)

    blk = 1000
    hw = pl.pallas_call(
        _mid_body,
        grid=(n // blk,),
        in_specs=[
            pl.BlockSpec((NC, blk, d), lambda i: (0, i, 0)),
            pl.BlockSpec((d, h_dim), lambda i: (0, 0)),
            pl.BlockSpec((1, h_dim), lambda i: (0, 0)),
            pl.BlockSpec((h_dim, o_dim), lambda i: (0, 0)),
        ],
        out_specs=pl.BlockSpec((blk, o_dim), lambda i: (i, 0)),
        out_shape=jax.ShapeDtypeStruct((n, o_dim), jnp.float32),
    )(p1, W0, b0.reshape(1, h_dim), W1)

    zeros_o = jnp.zeros((n, o_dim), jnp.float32)
    p2 = _make_segsum(n, e, o_dim)(hw, src, dst, zeros_o)

    out = pl.pallas_call(
        _final_body,
        grid=(n // blk,),
        in_specs=[
            pl.BlockSpec((NC, blk, o_dim), lambda i: (0, i, 0)),
            pl.BlockSpec((1, o_dim), lambda i: (0, 0)),
        ],
        out_specs=pl.BlockSpec((blk, o_dim), lambda i: (i, 0)),
        out_shape=jax.ShapeDtypeStruct((n, o_dim), jnp.float32),
    )(p2, b1.reshape(1, o_dim))
    return out


# speedup vs baseline: 14.4164x; 1.1101x over previous
import functools

import jax
import jax.numpy as jnp
from jax import lax
from jax.experimental import pallas as pl
from jax.experimental.pallas import tpu as pltpu
from jax.experimental.pallas import tpu_sc as plsc

NC = 2
NS = 16
NW = NC * NS

_ZROWS = 25


@functools.lru_cache(maxsize=None)
def _make_segsum(n, e, d, chunk, nbuf):
    assert e % (NW * chunk) == 0
    e_per = e // NW
    nch = e_per // chunk
    assert nch % nbuf == 0
    nit = nch // nbuf
    rps = n // NS
    assert rps % _ZROWS == 0
    mesh = plsc.VectorSubcoreMesh(
        core_axis_name="c", subcore_axis_name="s", num_cores=NC, num_subcores=NS
    )

    @functools.partial(
        pl.kernel,
        mesh=mesh,
        out_type=jax.ShapeDtypeStruct((NC, n, d), jnp.float32),
        scratch_types=[
            pltpu.VMEM((nch, chunk), jnp.int32),
            pltpu.VMEM((nch, chunk), jnp.int32),
            [pltpu.VMEM((chunk, d), jnp.float32) for _ in range(nbuf)],
            pltpu.VMEM((_ZROWS, d), jnp.float32),
            pltpu.VMEM_SHARED((n, d), jnp.float32),
            [pltpu.SemaphoreType.DMA for _ in range(nbuf)],
            [pltpu.SemaphoreType.DMA for _ in range(nbuf)],
        ],
        compiler_params=pltpu.CompilerParams(use_tc_tiling_on_sc=False),
    )
    def segsum(x_hbm, ei_hbm, out_hbm,
               src_v, dst_v, rows, zbuf, acc, sem_g, sem_s):
        c = lax.axis_index("c")
        s = lax.axis_index("s")
        wid = s * NC + c
        row0 = s * rps

        pltpu.sync_copy(ei_hbm.at[0, pl.ds(wid * nch, nch)], src_v)
        pltpu.sync_copy(ei_hbm.at[1, pl.ds(wid * nch, nch)], dst_v)

        def zrow(r, carry):
            def zcol(q, carry2):
                zbuf[r, pl.ds(q * 16, 16)] = jnp.zeros((16,), jnp.float32)
                return carry2
            return lax.fori_loop(0, d // 16, zcol, carry)

        lax.fori_loop(0, _ZROWS, zrow, 0)

        def zcopy(i, carry):
            pltpu.sync_copy(zbuf, acc.at[pl.ds(row0 + i * _ZROWS, _ZROWS)])
            return carry

        lax.fori_loop(0, rps // _ZROWS, zcopy, 0)
        plsc.subcore_barrier()

        def body(k, carry):
            j0 = k * nbuf
            for b in range(nbuf):
                @pl.when(k > 0)
                def _():
                    pltpu.make_async_copy(
                        rows[b], acc.at[pl.ds(0, chunk)], sem_s[b]).wait()
                pltpu.async_copy(x_hbm.at[src_v.at[j0 + b]], rows[b], sem_g[b])
            for b in range(nbuf):
                pltpu.make_async_copy(
                    x_hbm.at[pl.ds(0, chunk)], rows[b], sem_g[b]).wait()
                pltpu.async_copy(rows[b], acc.at[dst_v.at[j0 + b]],
                                 sem_s[b], add=True)
            return carry

        lax.fori_loop(0, nit, body, 0)
        for b in range(nbuf):
            pltpu.make_async_copy(
                rows[b], acc.at[pl.ds(0, chunk)], sem_s[b]).wait()
        plsc.subcore_barrier()
        pltpu.sync_copy(acc.at[pl.ds(row0, rps)],
                        out_hbm.at[c, pl.ds(row0, rps)])

    return segsum


def _mid_body(p_ref, w0_ref, b0_ref, w1_ref, o_ref):
    ssum = p_ref[0] + p_ref[1]
    h = jnp.dot(ssum, w0_ref[...], preferred_element_type=jnp.float32)
    h = jnp.maximum(h + b0_ref[...], 0.0)
    o_ref[...] = jnp.dot(h, w1_ref[...], preferred_element_type=jnp.float32)


def _final_body(p_ref, b1_ref, o_ref):
    ssum = p_ref[0] + p_ref[1] + b1_ref[...]
    m = jnp.max(ssum, axis=1, keepdims=True)
    shifted = ssum - m
    o_ref[...] = shifted - jnp.log(jnp.sum(jnp.exp(shifted), axis=1, keepdims=True))


def kernel(x, edge_index, W0, b0, W1, b1):
    n, d = x.shape
    h_dim = W0.shape[1]
    o_dim = W1.shape[1]
    e = edge_index.shape[1]

    ei128 = edge_index.reshape(2, e // 40, 40)
    p1 = _make_segsum(n, e, d, 40, 5)(x, ei128)

    blk = 1000
    hw = pl.pallas_call(
        _mid_body,
        grid=(n // blk,),
        in_specs=[
            pl.BlockSpec((NC, blk, d), lambda i: (0, i, 0)),
            pl.BlockSpec((d, h_dim), lambda i: (0, 0)),
            pl.BlockSpec((1, h_dim), lambda i: (0, 0)),
            pl.BlockSpec((h_dim, o_dim), lambda i: (0, 0)),
        ],
        out_specs=pl.BlockSpec((blk, o_dim), lambda i: (i, 0)),
        out_shape=jax.ShapeDtypeStruct((n, o_dim), jnp.float32),
    )(p1, W0, b0.reshape(1, h_dim), W1)

    ei64 = edge_index.reshape(2, e // 80, 80)
    p2 = _make_segsum(n, e, o_dim, 80, 5)(hw, ei64)

    out = pl.pallas_call(
        _final_body,
        grid=(n // blk,),
        in_specs=[
            pl.BlockSpec((NC, blk, o_dim), lambda i: (0, i, 0)),
            pl.BlockSpec((1, o_dim), lambda i: (0, 0)),
        ],
        out_specs=pl.BlockSpec((blk, o_dim), lambda i: (i, 0)),
        out_shape=jax.ShapeDtypeStruct((n, o_dim), jnp.float32),
    )(p2, b1.reshape(1, o_dim))
    return out


# speedup vs baseline: 14.8217x; 1.0281x over previous
import functools

import jax
import jax.numpy as jnp
from jax import lax
from jax.experimental import pallas as pl
from jax.experimental.pallas import tpu as pltpu
from jax.experimental.pallas import tpu_sc as plsc

NC = 2
NS = 16
NW = NC * NS

_ZROWS = 25


@functools.lru_cache(maxsize=None)
def _make_segsum(n, e, d, chunk, nbuf):
    assert e % (NW * chunk) == 0
    e_per = e // NW
    nch = e_per // chunk
    assert nch % nbuf == 0
    nit = nch // nbuf
    rps = n // NS
    assert rps % _ZROWS == 0
    mesh = plsc.VectorSubcoreMesh(
        core_axis_name="c", subcore_axis_name="s", num_cores=NC, num_subcores=NS
    )

    @functools.partial(
        pl.kernel,
        mesh=mesh,
        out_type=jax.ShapeDtypeStruct((NC, n, d), jnp.float32),
        scratch_types=[
            pltpu.VMEM((nch, chunk), jnp.int32),
            pltpu.VMEM((nch, chunk), jnp.int32),
            [pltpu.VMEM((chunk, d), jnp.float32) for _ in range(nbuf)],
            pltpu.VMEM((_ZROWS, d), jnp.float32),
            pltpu.VMEM_SHARED((n, d), jnp.float32),
            [pltpu.SemaphoreType.DMA for _ in range(nbuf)],
            [pltpu.SemaphoreType.DMA for _ in range(nbuf)],
        ],
        compiler_params=pltpu.CompilerParams(use_tc_tiling_on_sc=False),
    )
    def segsum(x_hbm, ei_hbm, out_hbm,
               src_v, dst_v, rows, zbuf, acc, sem_g, sem_s):
        c = lax.axis_index("c")
        s = lax.axis_index("s")
        wid = s * NC + c
        row0 = s * rps

        pltpu.async_copy(ei_hbm.at[0, pl.ds(wid * nch, nch)], src_v, sem_g[0])
        pltpu.async_copy(ei_hbm.at[1, pl.ds(wid * nch, nch)], dst_v, sem_g[1])

        def zrow(r, carry):
            def zcol(q, carry2):
                zbuf[r, pl.ds(q * 16, 16)] = jnp.zeros((16,), jnp.float32)
                return carry2
            return lax.fori_loop(0, d // 16, zcol, carry)

        lax.fori_loop(0, _ZROWS, zrow, 0)

        nz = rps // _ZROWS

        def zcopy(i, carry):
            pltpu.async_copy(zbuf, acc.at[pl.ds(row0 + i * _ZROWS, _ZROWS)],
                             sem_s[0])
            return carry

        lax.fori_loop(0, nz, zcopy, 0)

        def zdrain(i, carry):
            pltpu.make_async_copy(zbuf, acc.at[pl.ds(row0, _ZROWS)],
                                  sem_s[0]).wait()
            return carry

        lax.fori_loop(0, nz, zdrain, 0)
        pltpu.make_async_copy(ei_hbm.at[0, pl.ds(0, nch)], src_v,
                              sem_g[0]).wait()
        pltpu.make_async_copy(ei_hbm.at[0, pl.ds(0, nch)], dst_v,
                              sem_g[1]).wait()
        plsc.subcore_barrier()

        def body(k, carry):
            j0 = k * nbuf
            for b in range(nbuf):
                @pl.when(k > 0)
                def _():
                    pltpu.make_async_copy(
                        rows[b], acc.at[pl.ds(0, chunk)], sem_s[b]).wait()
                pltpu.async_copy(x_hbm.at[src_v.at[j0 + b]], rows[b], sem_g[b])
            for b in range(nbuf):
                pltpu.make_async_copy(
                    x_hbm.at[pl.ds(0, chunk)], rows[b], sem_g[b]).wait()
                pltpu.async_copy(rows[b], acc.at[dst_v.at[j0 + b]],
                                 sem_s[b], add=True)
            return carry

        lax.fori_loop(0, nit, body, 0)
        for b in range(nbuf):
            pltpu.make_async_copy(
                rows[b], acc.at[pl.ds(0, chunk)], sem_s[b]).wait()
        plsc.subcore_barrier()
        pltpu.sync_copy(acc.at[pl.ds(row0, rps)],
                        out_hbm.at[c, pl.ds(row0, rps)])

    return segsum


def _mid_body(p_ref, w0_ref, b0_ref, w1_ref, o_ref):
    ssum = p_ref[0] + p_ref[1]
    h = jnp.dot(ssum, w0_ref[...], preferred_element_type=jnp.float32)
    h = jnp.maximum(h + b0_ref[...], 0.0)
    o_ref[...] = jnp.dot(h, w1_ref[...], preferred_element_type=jnp.float32)


def _final_body(p_ref, b1_ref, o_ref):
    ssum = p_ref[0] + p_ref[1] + b1_ref[...]
    m = jnp.max(ssum, axis=1, keepdims=True)
    shifted = ssum - m
    o_ref[...] = shifted - jnp.log(jnp.sum(jnp.exp(shifted), axis=1, keepdims=True))


def kernel(x, edge_index, W0, b0, W1, b1):
    n, d = x.shape
    h_dim = W0.shape[1]
    o_dim = W1.shape[1]
    e = edge_index.shape[1]

    ei128 = edge_index.reshape(2, e // 40, 40)
    p1 = _make_segsum(n, e, d, 40, 5)(x, ei128)

    blk = 1000
    hw = pl.pallas_call(
        _mid_body,
        grid=(n // blk,),
        in_specs=[
            pl.BlockSpec((NC, blk, d), lambda i: (0, i, 0)),
            pl.BlockSpec((d, h_dim), lambda i: (0, 0)),
            pl.BlockSpec((1, h_dim), lambda i: (0, 0)),
            pl.BlockSpec((h_dim, o_dim), lambda i: (0, 0)),
        ],
        out_specs=pl.BlockSpec((blk, o_dim), lambda i: (i, 0)),
        out_shape=jax.ShapeDtypeStruct((n, o_dim), jnp.float32),
    )(p1, W0, b0.reshape(1, h_dim), W1)

    ei64 = edge_index.reshape(2, e // 80, 80)
    p2 = _make_segsum(n, e, o_dim, 80, 5)(hw, ei64)

    out = pl.pallas_call(
        _final_body,
        grid=(n // blk,),
        in_specs=[
            pl.BlockSpec((NC, blk, o_dim), lambda i: (0, i, 0)),
            pl.BlockSpec((1, o_dim), lambda i: (0, 0)),
        ],
        out_specs=pl.BlockSpec((blk, o_dim), lambda i: (i, 0)),
        out_shape=jax.ShapeDtypeStruct((n, o_dim), jnp.float32),
    )(p2, b1.reshape(1, o_dim))
    return out


# speedup vs baseline: 15.1417x; 1.0216x over previous
import functools

import jax
import jax.numpy as jnp
from jax import lax
from jax.experimental import pallas as pl
from jax.experimental.pallas import tpu as pltpu
from jax.experimental.pallas import tpu_sc as plsc

NC = 2
NS = 16
NW = NC * NS

_ZROWS = 25


@functools.lru_cache(maxsize=None)
def _make_segsum(n, e, d, chunk, nbuf):
    assert e % (NW * chunk) == 0
    e_per = e // NW
    nch = e_per // chunk
    assert nch % nbuf == 0
    nit = nch // nbuf
    rps = n // NS
    assert rps % _ZROWS == 0
    mesh = plsc.VectorSubcoreMesh(
        core_axis_name="c", subcore_axis_name="s", num_cores=NC, num_subcores=NS
    )

    @functools.partial(
        pl.kernel,
        mesh=mesh,
        out_type=jax.ShapeDtypeStruct((NC, n, d), jnp.float32),
        scratch_types=[
            pltpu.VMEM((nch, chunk), jnp.int32),
            pltpu.VMEM((nch, chunk), jnp.int32),
            [pltpu.VMEM((chunk, d), jnp.float32) for _ in range(nbuf)],
            pltpu.VMEM((_ZROWS, d), jnp.float32),
            pltpu.VMEM_SHARED((n, d), jnp.float32),
            [pltpu.SemaphoreType.DMA for _ in range(nbuf)],
            [pltpu.SemaphoreType.DMA for _ in range(nbuf)],
        ],
        compiler_params=pltpu.CompilerParams(
            use_tc_tiling_on_sc=False,
            disable_bounds_checks=True,
            disable_semaphore_checks=True,
        ),
    )
    def segsum(x_hbm, ei_hbm, out_hbm,
               src_v, dst_v, rows, zbuf, acc, sem_g, sem_s):
        c = lax.axis_index("c")
        s = lax.axis_index("s")
        wid = s * NC + c
        row0 = s * rps

        pltpu.async_copy(ei_hbm.at[0, pl.ds(wid * nch, nch)], src_v, sem_g[0])
        pltpu.async_copy(ei_hbm.at[1, pl.ds(wid * nch, nch)], dst_v, sem_g[1])

        def zrow(r, carry):
            def zcol(q, carry2):
                zbuf[r, pl.ds(q * 16, 16)] = jnp.zeros((16,), jnp.float32)
                return carry2
            return lax.fori_loop(0, d // 16, zcol, carry)

        lax.fori_loop(0, _ZROWS, zrow, 0)

        nz = rps // _ZROWS

        def zcopy(i, carry):
            pltpu.async_copy(zbuf, acc.at[pl.ds(row0 + i * _ZROWS, _ZROWS)],
                             sem_s[0])
            return carry

        lax.fori_loop(0, nz, zcopy, 0)

        def zdrain(i, carry):
            pltpu.make_async_copy(zbuf, acc.at[pl.ds(row0, _ZROWS)],
                                  sem_s[0]).wait()
            return carry

        lax.fori_loop(0, nz, zdrain, 0)
        pltpu.make_async_copy(ei_hbm.at[0, pl.ds(0, nch)], src_v,
                              sem_g[0]).wait()
        pltpu.make_async_copy(ei_hbm.at[0, pl.ds(0, nch)], dst_v,
                              sem_g[1]).wait()
        plsc.subcore_barrier()

        def body(k, carry):
            j0 = k * nbuf
            for b in range(nbuf):
                @pl.when(k > 0)
                def _():
                    pltpu.make_async_copy(
                        rows[b], acc.at[pl.ds(0, chunk)], sem_s[b]).wait()
                pltpu.async_copy(x_hbm.at[src_v.at[j0 + b]], rows[b], sem_g[b])
            for b in range(nbuf):
                pltpu.make_async_copy(
                    x_hbm.at[pl.ds(0, chunk)], rows[b], sem_g[b]).wait()
                pltpu.async_copy(rows[b], acc.at[dst_v.at[j0 + b]],
                                 sem_s[b], add=True)
            return carry

        lax.fori_loop(0, nit, body, 0)
        for b in range(nbuf):
            pltpu.make_async_copy(
                rows[b], acc.at[pl.ds(0, chunk)], sem_s[b]).wait()
        plsc.subcore_barrier()
        pltpu.sync_copy(acc.at[pl.ds(row0, rps)],
                        out_hbm.at[c, pl.ds(row0, rps)])

    return segsum


def _mid_body(p_ref, w0_ref, b0_ref, w1_ref, o_ref):
    ssum = p_ref[0] + p_ref[1]
    h = jnp.dot(ssum, w0_ref[...], preferred_element_type=jnp.float32)
    h = jnp.maximum(h + b0_ref[...], 0.0)
    o_ref[...] = jnp.dot(h, w1_ref[...], preferred_element_type=jnp.float32)


def _final_body(p_ref, b1_ref, o_ref):
    ssum = p_ref[0] + p_ref[1] + b1_ref[...]
    m = jnp.max(ssum, axis=1, keepdims=True)
    shifted = ssum - m
    o_ref[...] = shifted - jnp.log(jnp.sum(jnp.exp(shifted), axis=1, keepdims=True))


def kernel(x, edge_index, W0, b0, W1, b1):
    n, d = x.shape
    h_dim = W0.shape[1]
    o_dim = W1.shape[1]
    e = edge_index.shape[1]

    ei128 = edge_index.reshape(2, e // 40, 40)
    p1 = _make_segsum(n, e, d, 40, 5)(x, ei128)

    blk = 2000
    hw = pl.pallas_call(
        _mid_body,
        grid=(n // blk,),
        in_specs=[
            pl.BlockSpec((NC, blk, d), lambda i: (0, i, 0)),
            pl.BlockSpec((d, h_dim), lambda i: (0, 0)),
            pl.BlockSpec((1, h_dim), lambda i: (0, 0)),
            pl.BlockSpec((h_dim, o_dim), lambda i: (0, 0)),
        ],
        out_specs=pl.BlockSpec((blk, o_dim), lambda i: (i, 0)),
        out_shape=jax.ShapeDtypeStruct((n, o_dim), jnp.float32),
    )(p1, W0, b0.reshape(1, h_dim), W1)

    ei64 = edge_index.reshape(2, e // 80, 80)
    p2 = _make_segsum(n, e, o_dim, 80, 5)(hw, ei64)

    out = pl.pallas_call(
        _final_body,
        grid=(n // blk,),
        in_specs=[
            pl.BlockSpec((NC, blk, o_dim), lambda i: (0, i, 0)),
            pl.BlockSpec((1, o_dim), lambda i: (0, 0)),
        ],
        out_specs=pl.BlockSpec((blk, o_dim), lambda i: (i, 0)),
        out_shape=jax.ShapeDtypeStruct((n, o_dim), jnp.float32),
    )(p2, b1.reshape(1, o_dim))
    return out


# speedup vs baseline: 15.2669x; 1.0083x over previous
import functools

import jax
import jax.numpy as jnp
from jax import lax
from jax.experimental import pallas as pl
from jax.experimental.pallas import tpu as pltpu
from jax.experimental.pallas import tpu_sc as plsc

NC = 2
NS = 16
NW = NC * NS

_ZROWS = 25


@functools.lru_cache(maxsize=None)
def _make_segsum(n, e, d, chunk, nbuf):
    assert e % (NW * chunk) == 0
    e_per = e // NW
    nch = e_per // chunk
    assert nch % nbuf == 0
    nit = nch // nbuf
    rps = n // NS
    assert rps % _ZROWS == 0
    mesh = plsc.VectorSubcoreMesh(
        core_axis_name="c", subcore_axis_name="s", num_cores=NC, num_subcores=NS
    )

    @functools.partial(
        pl.kernel,
        mesh=mesh,
        out_type=jax.ShapeDtypeStruct((NC, n, d), jnp.float32),
        scratch_types=[
            pltpu.VMEM((nch, chunk), jnp.int32),
            pltpu.VMEM((nch, chunk), jnp.int32),
            [pltpu.VMEM((chunk, d), jnp.float32) for _ in range(nbuf)],
            pltpu.VMEM((_ZROWS, d), jnp.float32),
            pltpu.VMEM_SHARED((n, d), jnp.float32),
            [pltpu.SemaphoreType.DMA for _ in range(nbuf)],
            [pltpu.SemaphoreType.DMA for _ in range(nbuf)],
        ],
        compiler_params=pltpu.CompilerParams(
            use_tc_tiling_on_sc=False,
            disable_bounds_checks=True,
            disable_semaphore_checks=True,
        ),
    )
    def segsum(x_hbm, ei_hbm, out_hbm,
               src_v, dst_v, rows, zbuf, acc, sem_g, sem_s):
        c = lax.axis_index("c")
        s = lax.axis_index("s")
        wid = s * NC + c
        row0 = s * rps

        pltpu.async_copy(ei_hbm.at[0, pl.ds(wid * nch, nch)], src_v, sem_g[0])
        pltpu.async_copy(ei_hbm.at[1, pl.ds(wid * nch, nch)], dst_v, sem_g[1])

        def zrow(r, carry):
            def zcol(q, carry2):
                zbuf[r, pl.ds(q * 16, 16)] = jnp.zeros((16,), jnp.float32)
                return carry2
            return lax.fori_loop(0, d // 16, zcol, carry)

        lax.fori_loop(0, _ZROWS, zrow, 0)

        nz = rps // _ZROWS

        def zcopy(i, carry):
            pltpu.async_copy(zbuf, acc.at[pl.ds(row0 + i * _ZROWS, _ZROWS)],
                             sem_s[0])
            return carry

        lax.fori_loop(0, nz, zcopy, 0)

        def zdrain(i, carry):
            pltpu.make_async_copy(zbuf, acc.at[pl.ds(row0, _ZROWS)],
                                  sem_s[0]).wait()
            return carry

        pltpu.make_async_copy(ei_hbm.at[0, pl.ds(0, nch)], src_v,
                              sem_g[0]).wait()
        pltpu.make_async_copy(ei_hbm.at[0, pl.ds(0, nch)], dst_v,
                              sem_g[1]).wait()
        for b in range(nbuf):
            pltpu.async_copy(x_hbm.at[src_v.at[b]], rows[b], sem_g[b])

        lax.fori_loop(0, nz, zdrain, 0)
        plsc.subcore_barrier()

        def body(k, carry):
            j0 = k * nbuf
            for b in range(nbuf):
                pltpu.make_async_copy(
                    x_hbm.at[pl.ds(0, chunk)], rows[b], sem_g[b]).wait()
                pltpu.async_copy(rows[b], acc.at[dst_v.at[j0 + b]],
                                 sem_s[b], add=True)
            for b in range(nbuf):
                @pl.when(k < nit - 1)
                def _():
                    pltpu.make_async_copy(
                        rows[b], acc.at[pl.ds(0, chunk)], sem_s[b]).wait()
                    pltpu.async_copy(x_hbm.at[src_v.at[j0 + nbuf + b]],
                                     rows[b], sem_g[b])
            return carry

        lax.fori_loop(0, nit, body, 0)
        for b in range(nbuf):
            pltpu.make_async_copy(
                rows[b], acc.at[pl.ds(0, chunk)], sem_s[b]).wait()
        plsc.subcore_barrier()
        pltpu.sync_copy(acc.at[pl.ds(row0, rps)],
                        out_hbm.at[c, pl.ds(row0, rps)])

    return segsum


def _mid_body(p_ref, w0_ref, b0_ref, w1_ref, o_ref):
    ssum = p_ref[0] + p_ref[1]
    h = jnp.dot(ssum, w0_ref[...], preferred_element_type=jnp.float32)
    h = jnp.maximum(h + b0_ref[...], 0.0)
    o_ref[...] = jnp.dot(h, w1_ref[...], preferred_element_type=jnp.float32)


def _final_body(p_ref, b1_ref, o_ref):
    ssum = p_ref[0] + p_ref[1] + b1_ref[...]
    m = jnp.max(ssum, axis=1, keepdims=True)
    shifted = ssum - m
    o_ref[...] = shifted - jnp.log(jnp.sum(jnp.exp(shifted), axis=1, keepdims=True))


def kernel(x, edge_index, W0, b0, W1, b1):
    n, d = x.shape
    h_dim = W0.shape[1]
    o_dim = W1.shape[1]
    e = edge_index.shape[1]

    ei128 = edge_index.reshape(2, e // 40, 40)
    p1 = _make_segsum(n, e, d, 40, 5)(x, ei128)

    blk = 2000
    hw = pl.pallas_call(
        _mid_body,
        grid=(n // blk,),
        in_specs=[
            pl.BlockSpec((NC, blk, d), lambda i: (0, i, 0)),
            pl.BlockSpec((d, h_dim), lambda i: (0, 0)),
            pl.BlockSpec((1, h_dim), lambda i: (0, 0)),
            pl.BlockSpec((h_dim, o_dim), lambda i: (0, 0)),
        ],
        out_specs=pl.BlockSpec((blk, o_dim), lambda i: (i, 0)),
        out_shape=jax.ShapeDtypeStruct((n, o_dim), jnp.float32),
    )(p1, W0, b0.reshape(1, h_dim), W1)

    ei64 = edge_index.reshape(2, e // 80, 80)
    p2 = _make_segsum(n, e, o_dim, 80, 5)(hw, ei64)

    out = pl.pallas_call(
        _final_body,
        grid=(n // blk,),
        in_specs=[
            pl.BlockSpec((NC, blk, o_dim), lambda i: (0, i, 0)),
            pl.BlockSpec((1, o_dim), lambda i: (0, 0)),
        ],
        out_specs=pl.BlockSpec((blk, o_dim), lambda i: (i, 0)),
        out_shape=jax.ShapeDtypeStruct((n, o_dim), jnp.float32),
    )(p2, b1.reshape(1, o_dim))
    return out
